# baseline (speedup 1.0000x reference)
import functools

import jax
import jax.numpy as jnp
from jax import lax
from jax.experimental import pallas as pl
from jax.experimental.pallas import tpu as pltpu
from jax.experimental.pallas import tpu_sc as plsc

_NE = 256
_TT = 4
_PAD = _NE * _TT
_D = 64
_NC, _NS = 2, 16
_NW = _NC * _NS
_C = 512
_SUB = 128


def _ids_seq_body(sem_ref, tt_ref, mask_ref, out_ref):
    sem = sem_ref[...]
    tt = jnp.clip(tt_ref[...], 0, _TT - 1)
    ids = tt * _NE + sem
    invalid = ((ids > _PAD - 1) | (ids < 0)) & (sem != -1)
    ids = jnp.where(invalid, _PAD, ids)
    out_ref[...] = jnp.where(mask_ref[...], ids, _PAD)


def _ids_fut_body(sem_ref, tt_ref, out_ref):
    sem = sem_ref[...]
    tt = jnp.clip(tt_ref[...], 0, _TT - 1)
    ids = tt * _NE + sem
    invalid = ((ids > _PAD - 1) | (ids < 0)) & (sem != -1)
    out_ref[...] = jnp.where(invalid, _PAD, ids)


def _sc_embed(ids_seq, ids_fut, table):
    N1, N2 = ids_seq.shape[0], ids_fut.shape[0]
    n1, n2 = N1 // _NW, N2 // _NW
    G1, G2 = n1 // _C, n2 // _C

    mesh = plsc.VectorSubcoreMesh(core_axis_name="c", subcore_axis_name="s")

    @functools.partial(
        pl.kernel,
        mesh=mesh,
        compiler_params=pltpu.CompilerParams(use_tc_tiling_on_sc=False),
        out_type=(
            jax.ShapeDtypeStruct((N1, _D), jnp.float32),
            jax.ShapeDtypeStruct((N2, _D), jnp.float32),
        ),
        scratch_types=[
            pltpu.VMEM((_C,), jnp.int32),
            pltpu.VMEM((_C, _D), jnp.float32),
            pltpu.SemaphoreType.DMA,
        ],
    )
    def k(ids1_hbm, ids2_hbm, tab_hbm, out1_hbm, out2_hbm, idx_v, rows_v, sem):
        wid = lax.axis_index("s") * _NC + lax.axis_index("c")

        def run(ids_hbm, out_hbm, tok0, g):
            tok = tok0 + g * _C
            pltpu.sync_copy(ids_hbm.at[pl.ds(tok, _C)], idx_v)
            hs = []
            for j in range(0, _C, _SUB):
                hs.append(pltpu.async_copy(
                    tab_hbm.at[idx_v.at[pl.ds(j, _SUB)]],
                    rows_v.at[pl.ds(j, _SUB)], sem))
            for h in hs:
                h.wait()
            pltpu.sync_copy(rows_v, out_hbm.at[pl.ds(tok, _C)])

        def seq_body(g, carry):
            run(ids1_hbm, out1_hbm, wid * n1, g)
            return carry

        lax.fori_loop(0, G1, seq_body, 0)

        def fut_body(g, carry):
            run(ids2_hbm, out2_hbm, wid * n2, g)
            return carry

        lax.fori_loop(0, G2, fut_body, 0)

    return k(ids_seq, ids_fut, table)


def kernel(sem_ids, token_type_ids, seq_mask, sem_ids_fut, token_type_ids_fut,
           emb_table):
    B, L = sem_ids.shape
    FUT = sem_ids_fut.shape[1]
    N1, N2 = B * L, B * FUT
    r1, r2 = N1 // 128, N2 // 128

    nblk = 8
    ids1 = pl.pallas_call(
        _ids_seq_body,
        grid=(nblk,),
        in_specs=[pl.BlockSpec((r1 // nblk, 128), lambda i: (i, 0))] * 3,
        out_specs=pl.BlockSpec((r1 // nblk, 128), lambda i: (i, 0)),
        out_shape=jax.ShapeDtypeStruct((r1, 128), jnp.int32),
    )(sem_ids.reshape(r1, 128), token_type_ids.reshape(r1, 128),
      seq_mask.reshape(r1, 128))

    ids2 = pl.pallas_call(
        _ids_fut_body,
        grid=(1,),
        in_specs=[pl.BlockSpec((r2, 128), lambda i: (0, 0))] * 2,
        out_specs=pl.BlockSpec((r2, 128), lambda i: (0, 0)),
        out_shape=jax.ShapeDtypeStruct((r2, 128), jnp.int32),
    )(sem_ids_fut.reshape(r2, 128), token_type_ids_fut.reshape(r2, 128))

    seq_flat, fut_flat = _sc_embed(ids1.reshape(N1), ids2.reshape(N2),
                                   emb_table)
    return seq_flat.reshape(B, L, _D), fut_flat.reshape(B, FUT, _D)
